# baseline (speedup 1.0000x reference)
import functools
import jax
import jax.numpy as jnp
from jax import lax
from jax.experimental import pallas as pl
from jax.experimental.pallas import tpu as pltpu
from jax.experimental.pallas import tpu_sc as plsc

D = 64
SCALE = 8.0

NC = 2
NS = 16
NW = NC * NS
CHUNK = 128
B = 4096 * 200
B_PER_W = B // NW
NCHUNK = B_PER_W // CHUNK


def _emb_body(x_hbm, w_hbm, out_hbm, idx_v, rows_v, sem):
    wid = lax.axis_index("s") * NC + lax.axis_index("c")
    base = wid * B_PER_W
    pltpu.sync_copy(x_hbm.at[wid], idx_v)

    def chunk_body(c, _):
        pltpu.async_copy(w_hbm.at[idx_v.at[c]], rows_v, sem).wait()

        def scale_row(i, _):
            for j in range(D // 16):
                sl = pl.ds(j * 16, 16)
                rows_v[i, sl] = rows_v[i, sl] * SCALE
            return 0

        lax.fori_loop(0, CHUNK, scale_row, 0)
        pltpu.sync_copy(rows_v, out_hbm.at[pl.ds(base + c * CHUNK, CHUNK)])
        return 0

    lax.fori_loop(0, NCHUNK, chunk_body, 0)


@functools.partial(jax.jit, static_argnames=())
def kernel(x, W):
    batch, seq = x.shape
    x_r = x.reshape(NW, NCHUNK, CHUNK).astype(jnp.int32)
    mesh = plsc.VectorSubcoreMesh(core_axis_name="c", subcore_axis_name="s")
    out = pl.kernel(
        _emb_body,
        mesh=mesh,
        compiler_params=pltpu.CompilerParams(use_tc_tiling_on_sc=False),
        out_type=jax.ShapeDtypeStruct((B, D), jnp.float32),
        scratch_types=[
            pltpu.VMEM((NCHUNK, CHUNK), jnp.int32),
            pltpu.VMEM((CHUNK, D), jnp.float32),
            pltpu.SemaphoreType.DMA,
        ],
    )(x_r, W)
    return out.reshape(batch, seq, D)


# speedup vs baseline: 1.2063x; 1.2063x over previous
import functools
import jax
import jax.numpy as jnp
from jax import lax
from jax.experimental import pallas as pl
from jax.experimental.pallas import tpu as pltpu
from jax.experimental.pallas import tpu_sc as plsc

D = 64
SCALE = 8.0

NC = 2
NS = 16
NW = NC * NS
CHUNK = 128
B = 4096 * 200
B_PER_W = B // NW
NCHUNK = B_PER_W // CHUNK

K = 2
ROWS_G = K * CHUNK
NBUF = 4
NG = NCHUNK // K


def _emb_body(x_hbm, w_hbm, out_hbm, idx_v, rows_v, gsem, wsem):
    wid = lax.axis_index("s") * NC + lax.axis_index("c")
    base = wid * B_PER_W
    pltpu.sync_copy(x_hbm.at[wid], idx_v)

    def fire_gathers(g, p):
        for j in range(K):
            pltpu.async_copy(
                w_hbm.at[idx_v.at[g * K + j]],
                rows_v.at[p, pl.ds(j * CHUNK, CHUNK)],
                gsem.at[p])

    def drain_gathers(g, p):
        pltpu.make_async_copy(
            out_hbm.at[pl.ds(base + g * ROWS_G, ROWS_G)],
            rows_v.at[p], gsem.at[p]).wait()

    def fire_wb(g, p):
        pltpu.async_copy(
            rows_v.at[p],
            out_hbm.at[pl.ds(base + g * ROWS_G, ROWS_G)],
            wsem.at[p])

    def wait_wb(g, p):
        pltpu.make_async_copy(
            out_hbm.at[pl.ds(base + g * ROWS_G, ROWS_G)],
            rows_v.at[p], wsem.at[p]).wait()

    def scale_buf(p):
        def row(i, _):
            for j in range(D // 16):
                sl = pl.ds(j * 16, 16)
                rows_v[p, i, sl] = rows_v[p, i, sl] * SCALE
            return 0
        lax.fori_loop(0, ROWS_G, row, 0)

    for r in range(NBUF - 1):
        fire_gathers(r, r)

    def step(i, _):
        for r in range(NBUF):
            g = i * NBUF + r
            drain_gathers(g, r)
            scale_buf(r)
            gn = g + NBUF - 1
            q = (r + NBUF - 1) % NBUF

            @pl.when(gn < NG)
            def _fire_ahead():
                @pl.when(g >= 1)
                def _wait_prev_wb():
                    wait_wb(g - 1, q)
                fire_gathers(gn, q)

            fire_wb(g, r)
        return 0

    lax.fori_loop(0, NG // NBUF, step, 0)

    for k in range(NBUF):
        gg = NG - NBUF + k
        wait_wb(gg, gg % NBUF)


@functools.partial(jax.jit, static_argnames=())
def kernel(x, W):
    batch, seq = x.shape
    x_r = x.reshape(NW, NCHUNK, CHUNK).astype(jnp.int32)
    mesh = plsc.VectorSubcoreMesh(core_axis_name="c", subcore_axis_name="s")
    out = pl.kernel(
        _emb_body,
        mesh=mesh,
        compiler_params=pltpu.CompilerParams(use_tc_tiling_on_sc=False),
        out_type=jax.ShapeDtypeStruct((B, D), jnp.float32),
        scratch_types=[
            pltpu.VMEM((NCHUNK, CHUNK), jnp.int32),
            pltpu.VMEM((NBUF, ROWS_G, D), jnp.float32),
            pltpu.SemaphoreType.DMA((NBUF,)),
            pltpu.SemaphoreType.DMA((NBUF,)),
        ],
    )(x_r, W)
    return out.reshape(batch, seq, D)
